# Initial kernel scaffold; baseline (speedup 1.0000x reference)
import jax
import jax.numpy as jnp
from jax.experimental import pallas as pl


def kernel(users, items, neg_items, U_mlp, U_mf, V_mlp, V_mf, U_mlp_g, U_mf_g, W1, b1, W2, b2):
    raise NotImplementedError("write your pallas kernel here")


# speedup vs baseline: 2.7168x; 2.7168x over previous
import functools

import jax
import jax.numpy as jnp
from jax import lax
from jax.experimental import pallas as pl
from jax.experimental.pallas import tpu as pltpu
from jax.experimental.pallas import tpu_sc as plsc

B = 4096
EMB = 128
NNEG = 50
NC, NS = 2, 16
NW = NC * NS
UPW = B // NW
RPW = B * NNEG // NW
CHUNK = 128
NCHUNK = RPW // CHUNK

_f32 = jnp.float32


def _sc_gather(users, items, neg_idx, U_mlp, U_mf, U_mlp_g, U_mf_g, V_mlp, V_mf):
    mesh = plsc.VectorSubcoreMesh(core_axis_name="c", subcore_axis_name="s")
    out_type = (
        jax.ShapeDtypeStruct((B, EMB), _f32),
        jax.ShapeDtypeStruct((B, EMB), _f32),
        jax.ShapeDtypeStruct((B, EMB), _f32),
        jax.ShapeDtypeStruct((B, EMB), _f32),
        jax.ShapeDtypeStruct((B, EMB), _f32),
        jax.ShapeDtypeStruct((B, EMB), _f32),
        jax.ShapeDtypeStruct((B * NNEG, EMB), _f32),
        jax.ShapeDtypeStruct((B * NNEG, EMB), _f32),
    )

    @functools.partial(
        pl.kernel,
        mesh=mesh,
        out_type=out_type,
        scratch_types=[
            pltpu.VMEM((UPW,), jnp.int32),
            pltpu.VMEM((NCHUNK, CHUNK), jnp.int32),
            pltpu.VMEM((UPW, EMB), _f32),
            pltpu.VMEM((CHUNK, EMB), _f32),
            pltpu.VMEM((CHUNK, EMB), _f32),
            pltpu.SemaphoreType.DMA,
        ],
    )
    def k(users_h, items_h, negidx_h, Umlp_h, Umf_h, Umlpg_h, Umfg_h, Vmlp_h, Vmf_h,
          umlp_o, umf_o, umlpg_o, umfg_o, vmlp_o, vmf_o, negmlp_o, negmf_o,
          idx_v, negidx_v, rows_v, bufa, bufb, sem):
        wid = lax.axis_index("s") * NC + lax.axis_index("c")
        ubase = wid * UPW
        pltpu.sync_copy(users_h.at[pl.ds(ubase, UPW)], idx_v)
        for tbl, out in ((Umlp_h, umlp_o), (Umf_h, umf_o),
                         (Umlpg_h, umlpg_o), (Umfg_h, umfg_o)):
            pltpu.async_copy(tbl.at[idx_v], rows_v, sem).wait()
            pltpu.sync_copy(rows_v, out.at[pl.ds(ubase, UPW)])
        pltpu.sync_copy(items_h.at[pl.ds(ubase, UPW)], idx_v)
        for tbl, out in ((Vmlp_h, vmlp_o), (Vmf_h, vmf_o)):
            pltpu.async_copy(tbl.at[idx_v], rows_v, sem).wait()
            pltpu.sync_copy(rows_v, out.at[pl.ds(ubase, UPW)])
        pltpu.sync_copy(negidx_h.at[wid], negidx_v)
        rbase = wid * RPW

        def chunk_body(c, carry):
            row0 = rbase + c * CHUNK
            pltpu.async_copy(Vmlp_h.at[negidx_v.at[c]], bufa, sem).wait()
            pltpu.sync_copy(bufa, negmlp_o.at[pl.ds(row0, CHUNK)])
            pltpu.async_copy(Vmf_h.at[negidx_v.at[c]], bufb, sem).wait()
            pltpu.sync_copy(bufb, negmf_o.at[pl.ds(row0, CHUNK)])
            return carry

        lax.fori_loop(0, NCHUNK, chunk_body, 0)

    return k(users, items, neg_idx, U_mlp, U_mf, U_mlp_g, U_mf_g, V_mlp, V_mf)


UB = 64


def _tc_compute(u_mlp, u_mf, v_mlp, v_mf, neg_v_mlp, neg_v_mf, W1, b1, W2, b2):
    def body(umlp_r, umf_r, vmlp_r, vmf_r, nvmlp_r, nvmf_r,
             W1_r, b1_r, W2_r, b2_r,
             mlp_o, mf_o, negmlp_o, negmf_o):
        W1u = W1_r[:EMB, :]
        W1v = W1_r[EMB:, :]
        b1 = b1_r[...]
        b2 = b2_r[...]
        W2 = W2_r[...]
        u = umlp_r[...]
        pre_u = jnp.dot(u, W1u, preferred_element_type=_f32) + b1
        h = jnp.maximum(
            pre_u + jnp.dot(vmlp_r[...], W1v, preferred_element_type=_f32), 0.0)
        mlp_o[...] = jnp.dot(h, W2, preferred_element_type=_f32) + b2
        mf_o[...] = umf_r[...] * vmf_r[...]
        nv = nvmlp_r[...].reshape(UB * NNEG, EMB)
        pre_e = jnp.broadcast_to(
            pre_u[:, None, :], (UB, NNEG, EMB)).reshape(UB * NNEG, EMB)
        hn = jnp.maximum(
            pre_e + jnp.dot(nv, W1v, preferred_element_type=_f32), 0.0)
        negmlp_o[...] = (jnp.dot(hn, W2, preferred_element_type=_f32)
                         + b2).reshape(UB, NNEG, EMB)
        negmf_o[...] = umf_r[...][:, None, :] * nvmf_r[...]

    grid = (B // UB,)
    vec2 = pl.BlockSpec((UB, EMB), lambda i: (i, 0))
    neg3 = pl.BlockSpec((UB, NNEG, EMB), lambda i: (i, 0, 0))
    full = lambda shape: pl.BlockSpec(shape, lambda i: tuple(0 for _ in shape))
    return pl.pallas_call(
        body,
        grid=grid,
        in_specs=[vec2, vec2, vec2, vec2, neg3, neg3,
                  full((2 * EMB, EMB)), full((1, EMB)),
                  full((EMB, EMB)), full((1, EMB))],
        out_specs=[vec2, vec2, neg3, neg3],
        out_shape=[
            jax.ShapeDtypeStruct((B, EMB), _f32),
            jax.ShapeDtypeStruct((B, EMB), _f32),
            jax.ShapeDtypeStruct((B, NNEG, EMB), _f32),
            jax.ShapeDtypeStruct((B, NNEG, EMB), _f32),
        ],
        compiler_params=pltpu.CompilerParams(
            dimension_semantics=("parallel",)),
    )(u_mlp, u_mf, v_mlp, v_mf, neg_v_mlp, neg_v_mf, W1, b1, W2, b2)


def kernel(users, items, neg_items, U_mlp, U_mf, V_mlp, V_mf, U_mlp_g, U_mf_g,
           W1, b1, W2, b2):
    users = users.astype(jnp.int32)
    items = items.astype(jnp.int32)
    neg_idx = neg_items.astype(jnp.int32).reshape(NW, NCHUNK, CHUNK)

    (u_mlp, u_mf, u_mlp_g, u_mf_g, v_mlp, v_mf,
     negmlp_flat, negmf_flat) = _sc_gather(
        users, items, neg_idx, U_mlp, U_mf, U_mlp_g, U_mf_g, V_mlp, V_mf)

    neg_v_mlp = negmlp_flat.reshape(B, NNEG, EMB)
    neg_v_mf = negmf_flat.reshape(B, NNEG, EMB)

    mlp_vector, mf_vector, neg_mlp_vector, neg_mf_vector = _tc_compute(
        u_mlp, u_mf, v_mlp, v_mf, neg_v_mlp, neg_v_mf,
        W1, b1.reshape(1, EMB), W2, b2.reshape(1, EMB))

    return (mlp_vector, mf_vector, u_mlp, u_mf, u_mlp_g, u_mf_g,
            neg_mlp_vector, neg_mf_vector)
